# baseline (speedup 1.0000x reference)
import jax
import jax.numpy as jnp
from jax.experimental import pallas as pl
from jax.experimental.pallas import tpu as pltpu

_B = 16384
_F = 100
_NF = 26
_D = 16
_FD = _F * _D
_BT = 1024


def _fm_kernel(x_ref, w_ref, v_ref, fi_ref, yfm_ref, out_ref, p_ref, a_ref):
    @pl.when(pl.program_id(0) == 0)
    def _init():
        fi = fi_ref[...]
        k_iota = jax.lax.broadcasted_iota(jnp.int32, (_F, _NF), 1)
        onehot = (fi == k_iota.astype(jnp.float32))
        onehot = onehot.astype(jnp.float32)
        embeds = jnp.dot(onehot, v_ref[...],
                         preferred_element_type=jnp.float32)
        d_iota = jax.lax.broadcasted_iota(jnp.int32, (_D, _FD), 0)
        j_iota = jax.lax.broadcasted_iota(jnp.int32, (_D, _FD), 1)
        tm = (j_iota % _D == d_iota).astype(jnp.float32)
        emb_b = jnp.dot(embeds, tm,
                        preferred_element_type=jnp.float32)
        f_iota = jax.lax.broadcasted_iota(jnp.int32, (_F, _FD), 0)
        jf = jax.lax.broadcasted_iota(jnp.int32, (_F, _FD), 1) // _D
        p_ref[...] = jnp.where(f_iota == jf, emb_b, 0.0).astype(jnp.bfloat16)
        ones_d = jnp.ones((_D, 1), jnp.float32)
        esum = jnp.dot(embeds, ones_d, preferred_element_type=jnp.float32)
        esq = jnp.dot(embeds * embeds, ones_d,
                      preferred_element_type=jnp.float32)
        a_ref[...] = jnp.concatenate([esum, esq], axis=1)

    x = x_ref[...]
    out_ref[...] = jnp.dot(x.astype(jnp.bfloat16), p_ref[...],
                           preferred_element_type=jnp.float32)
    lin = jnp.sum(x * w_ref[...], axis=1, keepdims=True)
    es = a_ref[...]
    s = jnp.dot(x, es[:, 0:1], preferred_element_type=jnp.float32)
    q = jnp.dot(x * x, es[:, 1:2], preferred_element_type=jnp.float32)
    inter = 0.5 * (s * s - q)
    yfm_ref[...] = jnp.concatenate([lin, inter], axis=1)


@jax.jit
def kernel(inputs, w, V, field_index):
    fi_f = field_index.astype(jnp.float32).reshape(_F, 1)
    w_row = w.reshape(1, _F)
    yfm, out2d = pl.pallas_call(
        _fm_kernel,
        grid=(_B // _BT,),
        in_specs=[
            pl.BlockSpec((_BT, _F), lambda i: (i, 0)),
            pl.BlockSpec((1, _F), lambda i: (0, 0)),
            pl.BlockSpec((_NF, _D), lambda i: (0, 0)),
            pl.BlockSpec((_F, 1), lambda i: (0, 0)),
        ],
        out_specs=[
            pl.BlockSpec((_BT, 2), lambda i: (i, 0)),
            pl.BlockSpec((_BT, _FD), lambda i: (i, 0)),
        ],
        out_shape=[
            jax.ShapeDtypeStruct((_B, 2), jnp.float32),
            jax.ShapeDtypeStruct((_B, _FD), jnp.float32),
        ],
        scratch_shapes=[
            pltpu.VMEM((_F, _FD), jnp.bfloat16),
            pltpu.VMEM((_F, 2), jnp.float32),
        ],
        compiler_params=pltpu.CompilerParams(
            dimension_semantics=("arbitrary",),
        ),
    )(inputs, w_row, V, fi_f)
    return yfm, out2d.reshape(_B, _F, _D)


# speedup vs baseline: 3.3832x; 3.3832x over previous
import jax
import jax.numpy as jnp
from jax.experimental import pallas as pl
from jax.experimental.pallas import tpu as pltpu

_B = 16384
_F = 100
_NF = 26
_D = 16
_FD = _F * _D
_BT = 2048


def _fm_kernel(x_ref, w_ref, vt_ref, fi_ref, yfm_ref, out_ref, pt_ref, a_ref):
    @pl.when(pl.program_id(0) == 0)
    def _init():
        fi = fi_ref[...]
        k_iota = jax.lax.broadcasted_iota(jnp.int32, (_NF, _F), 0)
        onehot_t = (fi == k_iota.astype(jnp.float32)).astype(jnp.float32)
        embeds_t = jnp.dot(vt_ref[...], onehot_t,
                           preferred_element_type=jnp.float32)
        j_iota = jax.lax.broadcasted_iota(jnp.int32, (_FD, _D), 0)
        d_iota = jax.lax.broadcasted_iota(jnp.int32, (_FD, _D), 1)
        tm_t = (j_iota % _D == d_iota).astype(jnp.float32)
        emb_rows = jnp.dot(tm_t, embeds_t,
                           preferred_element_type=jnp.float32)
        jf = jax.lax.broadcasted_iota(jnp.int32, (_FD, _F), 0) // _D
        f_iota = jax.lax.broadcasted_iota(jnp.int32, (_FD, _F), 1)
        pt_ref[...] = jnp.where(jf == f_iota, emb_rows, 0.0).astype(jnp.bfloat16)
        esum = jnp.sum(embeds_t, axis=0, keepdims=True)
        esq = jnp.sum(embeds_t * embeds_t, axis=0, keepdims=True)
        a_ref[...] = jnp.concatenate([esum, esq], axis=0)

    xt = jnp.transpose(x_ref[...], (1, 0))
    out_ref[...] = jnp.dot(pt_ref[...], xt.astype(jnp.bfloat16),
                           preferred_element_type=jnp.float32)
    ws = jnp.concatenate([w_ref[...], a_ref[0:1, :]], axis=0)
    m1 = jnp.dot(ws, xt, preferred_element_type=jnp.float32)
    q = jnp.dot(a_ref[1:2, :], xt * xt,
                preferred_element_type=jnp.float32)
    inter = 0.5 * (m1[1:2, :] * m1[1:2, :] - q)
    yfm_ref[...] = jnp.concatenate([m1[0:1, :], inter], axis=0)


@jax.jit
def kernel(inputs, w, V, field_index):
    fi_row = field_index.astype(jnp.float32).reshape(1, _F)
    w_row = w.reshape(1, _F)
    v_t = V.T
    yfm_t, out_t = pl.pallas_call(
        _fm_kernel,
        grid=(_B // _BT,),
        in_specs=[
            pl.BlockSpec((_BT, _F), lambda i: (i, 0)),
            pl.BlockSpec((1, _F), lambda i: (0, 0)),
            pl.BlockSpec((_D, _NF), lambda i: (0, 0)),
            pl.BlockSpec((1, _F), lambda i: (0, 0)),
        ],
        out_specs=[
            pl.BlockSpec((2, _BT), lambda i: (0, i)),
            pl.BlockSpec((_FD, _BT), lambda i: (0, i)),
        ],
        out_shape=[
            jax.ShapeDtypeStruct((2, _B), jnp.float32),
            jax.ShapeDtypeStruct((_FD, _B), jnp.float32),
        ],
        scratch_shapes=[
            pltpu.VMEM((_FD, _F), jnp.bfloat16),
            pltpu.VMEM((2, _F), jnp.float32),
        ],
        compiler_params=pltpu.CompilerParams(
            dimension_semantics=("arbitrary",),
        ),
    )(inputs, w_row, v_t, fi_row)
    y_fm = yfm_t.T
    new_inputs = out_t.reshape(_F, _D, _B).transpose(2, 0, 1)
    return y_fm, new_inputs


# speedup vs baseline: 4.2120x; 1.2450x over previous
import jax
import jax.numpy as jnp
from jax.experimental import pallas as pl
from jax.experimental.pallas import tpu as pltpu

_B = 16384
_F = 100
_NF = 26
_D = 16
_FD = _F * _D
_BT = 2048


def _fm_kernel(x_ref, w_ref, vt_ref, fi_ref, yfm_ref, out_ref, pt_ref, a_ref):
    @pl.when(pl.program_id(0) == 0)
    def _init():
        fi = fi_ref[...]
        k_iota = jax.lax.broadcasted_iota(jnp.int32, (_NF, _F), 0)
        onehot_t = (fi == k_iota.astype(jnp.float32)).astype(jnp.float32)
        embeds_t = jnp.dot(vt_ref[...], onehot_t,
                           preferred_element_type=jnp.float32)
        j_iota = jax.lax.broadcasted_iota(jnp.int32, (_FD, _D), 0)
        d_iota = jax.lax.broadcasted_iota(jnp.int32, (_FD, _D), 1)
        tm_t = (j_iota % _D == d_iota).astype(jnp.float32)
        emb_rows = jnp.dot(tm_t, embeds_t,
                           preferred_element_type=jnp.float32)
        jf = jax.lax.broadcasted_iota(jnp.int32, (_FD, _F), 0) // _D
        f_iota = jax.lax.broadcasted_iota(jnp.int32, (_FD, _F), 1)
        pt_ref[...] = jnp.where(jf == f_iota, emb_rows, 0.0).astype(jnp.bfloat16)
        esum = jnp.sum(embeds_t, axis=0, keepdims=True)
        esq = jnp.sum(embeds_t * embeds_t, axis=0, keepdims=True)
        a_ref[...] = jnp.concatenate([esum, esq], axis=0)

    xt = x_ref[...]
    out_ref[...] = jnp.dot(pt_ref[...], xt.astype(jnp.bfloat16),
                           preferred_element_type=jnp.float32)
    ws = jnp.concatenate([w_ref[...], a_ref[0:1, :]], axis=0)
    m1 = jnp.dot(ws, xt, preferred_element_type=jnp.float32)
    q = jnp.dot(a_ref[1:2, :], xt * xt,
                preferred_element_type=jnp.float32)
    inter = 0.5 * (m1[1:2, :] * m1[1:2, :] - q)
    yfm_ref[...] = jnp.concatenate([m1[0:1, :], inter], axis=0)


@jax.jit
def kernel(inputs, w, V, field_index):
    fi_row = field_index.astype(jnp.float32).reshape(1, _F)
    w_row = w.reshape(1, _F)
    v_t = V.T
    x_t = inputs.T
    yfm_t, out_t = pl.pallas_call(
        _fm_kernel,
        grid=(_B // _BT,),
        in_specs=[
            pl.BlockSpec((_F, _BT), lambda i: (0, i)),
            pl.BlockSpec((1, _F), lambda i: (0, 0)),
            pl.BlockSpec((_D, _NF), lambda i: (0, 0)),
            pl.BlockSpec((1, _F), lambda i: (0, 0)),
        ],
        out_specs=[
            pl.BlockSpec((2, _BT), lambda i: (0, i)),
            pl.BlockSpec((_FD, _BT), lambda i: (0, i)),
        ],
        out_shape=[
            jax.ShapeDtypeStruct((2, _B), jnp.float32),
            jax.ShapeDtypeStruct((_FD, _B), jnp.float32),
        ],
        scratch_shapes=[
            pltpu.VMEM((_FD, _F), jnp.bfloat16),
            pltpu.VMEM((2, _F), jnp.float32),
        ],
        compiler_params=pltpu.CompilerParams(
            dimension_semantics=("arbitrary",),
        ),
    )(x_t, w_row, v_t, fi_row)
    y_fm = yfm_t.T
    new_inputs = out_t.reshape(_F, _D, _B).transpose(2, 0, 1)
    return y_fm, new_inputs
